# baseline (speedup 1.0000x reference)
import functools

import jax
import jax.numpy as jnp
from jax.experimental import pallas as pl
from jax.experimental.pallas import tpu as pltpu

_LANES = 512
_SUB = 8
_UNROLL = 2


def _iou_chunk(a, b, mask):
    r2a = pltpu.roll(a, _LANES - 2, 1)
    r2b = pltpu.roll(b, _LANES - 2, 1)
    r3a = pltpu.roll(r2a, _LANES - 1, 1)
    r3b = pltpu.roll(r2b, _LANES - 1, 1)
    lt = jnp.maximum(a, b)
    rb = jnp.minimum(a + r2a, b + r2b)
    wh = jnp.maximum(rb - lt, 0.0)
    inter = wh * pltpu.roll(wh, _LANES - 1, 1)
    denom = r2a * r3a + r2b * r3b - inter
    iou = inter / jnp.maximum(denom, 1e-30)
    return jnp.where(mask, iou, 0.0)


def _iou_body(a_ref, b_ref, o_ref, acc_ref, *, steps, block_rows):
    j = pl.program_id(1)
    rows_per_iter = _SUB * _UNROLL
    lane = jax.lax.broadcasted_iota(jnp.int32, (_SUB, _LANES), 1)
    mask = (lane & 3) == 0

    def body(i, acc):
        new = []
        for k in range(_UNROLL):
            base = pl.multiple_of(i * rows_per_iter + k * _SUB, _SUB)
            a = a_ref[pl.ds(base, _SUB), :]
            b = b_ref[pl.ds(base, _SUB), :]
            new.append(_iou_chunk(a, b, mask))
        for v in new:
            acc = acc + v
        return acc

    acc = jax.lax.fori_loop(
        0, block_rows // rows_per_iter, body,
        jnp.zeros((_SUB, _LANES), jnp.float32))

    @pl.when(j == 0)
    def _():
        acc_ref[...] = acc

    @pl.when(j > 0)
    def _():
        acc_ref[...] = acc_ref[...] + acc

    @pl.when(j == steps - 1)
    def _():
        o_ref[...] = jnp.sum(acc_ref[...], axis=0, keepdims=True).reshape(
            1, 1, _LANES)


@jax.jit
def kernel(boxes_pred, boxes):
    n = boxes_pred.shape[0]
    rows = (n * 4) // _LANES
    a2 = boxes_pred.reshape(rows, _LANES)
    b2 = boxes.reshape(rows, _LANES)
    cores = 2
    block_rows = 512
    steps = rows // (cores * block_rows)
    body = functools.partial(_iou_body, steps=steps, block_rows=block_rows)
    partials = pl.pallas_call(
        body,
        grid=(cores, steps),
        in_specs=[
            pl.BlockSpec((block_rows, _LANES),
                         lambda i, j, s=steps: (i * s + j, 0)),
            pl.BlockSpec((block_rows, _LANES),
                         lambda i, j, s=steps: (i * s + j, 0)),
        ],
        out_specs=pl.BlockSpec((1, 1, _LANES), lambda i, j: (i, 0, 0)),
        out_shape=jax.ShapeDtypeStruct((cores, 1, _LANES), jnp.float32),
        scratch_shapes=[pltpu.VMEM((_SUB, _LANES), jnp.float32)],
        compiler_params=pltpu.CompilerParams(
            dimension_semantics=("parallel", "arbitrary"),
        ),
    )(a2, b2)
    return -jnp.log(jnp.sum(partials) / n)


# speedup vs baseline: 88.4312x; 88.4312x over previous
import functools

import jax
import jax.numpy as jnp
from jax.experimental import pallas as pl
from jax.experimental.pallas import tpu as pltpu

_L = 128
_SUB = 16
_UNROLL = 2


def _iou_chunk(a, b, mask):
    rows = a.shape[0]
    r2a = pltpu.roll(a, rows - 2, 0)
    r2b = pltpu.roll(b, rows - 2, 0)
    r3a = pltpu.roll(r2a, rows - 1, 0)
    r3b = pltpu.roll(r2b, rows - 1, 0)
    lt = jnp.maximum(a, b)
    rb = jnp.minimum(a + r2a, b + r2b)
    wh = jnp.maximum(rb - lt, 0.0)
    inter = wh * pltpu.roll(wh, rows - 1, 0)
    denom = r2a * r3a + r2b * r3b - inter
    iou = inter / jnp.maximum(denom, 1e-30)
    return jnp.where(mask, iou, 0.0)


def _iou_body(a_ref, b_ref, o_ref, acc_ref, *, steps, block_groups):
    j = pl.program_id(1)
    groups_per_iter = _SUB * _UNROLL
    row = jax.lax.broadcasted_iota(jnp.int32, (_SUB * 4, _L), 0)
    mask = (row & 3) == 0

    def body(i, acc):
        new = []
        for k in range(_UNROLL):
            base = pl.multiple_of(i * groups_per_iter + k * _SUB, _SUB)
            a = a_ref[pl.ds(base, _SUB), :, :].reshape(_SUB * 4, _L)
            b = b_ref[pl.ds(base, _SUB), :, :].reshape(_SUB * 4, _L)
            new.append(_iou_chunk(a, b, mask))
        for v in new:
            acc = acc + v
        return acc

    acc = jax.lax.fori_loop(
        0, block_groups // groups_per_iter, body,
        jnp.zeros((_SUB * 4, _L), jnp.float32))

    @pl.when(j == 0)
    def _():
        acc_ref[...] = acc

    @pl.when(j > 0)
    def _():
        acc_ref[...] = acc_ref[...] + acc

    @pl.when(j == steps - 1)
    def _():
        o_ref[...] = jnp.sum(acc_ref[...], axis=0, keepdims=True).reshape(
            1, 1, _L)


@jax.jit
def kernel(boxes_pred, boxes):
    n = boxes_pred.shape[0]
    groups = n // _L
    a3 = boxes_pred.T.reshape(4, groups, _L).transpose(1, 0, 2)
    b3 = boxes.T.reshape(4, groups, _L).transpose(1, 0, 2)
    cores = 2
    block_groups = 2048
    steps = groups // (cores * block_groups)
    body = functools.partial(_iou_body, steps=steps,
                             block_groups=block_groups)
    partials = pl.pallas_call(
        body,
        grid=(cores, steps),
        in_specs=[
            pl.BlockSpec((block_groups, 4, _L),
                         lambda i, j, s=steps: (i * s + j, 0, 0)),
            pl.BlockSpec((block_groups, 4, _L),
                         lambda i, j, s=steps: (i * s + j, 0, 0)),
        ],
        out_specs=pl.BlockSpec((1, 1, _L), lambda i, j: (i, 0, 0)),
        out_shape=jax.ShapeDtypeStruct((cores, 1, _L), jnp.float32),
        scratch_shapes=[pltpu.VMEM((_SUB * 4, _L), jnp.float32)],
        compiler_params=pltpu.CompilerParams(
            dimension_semantics=("parallel", "arbitrary"),
        ),
    )(a3, b3)
    return -jnp.log(jnp.sum(partials) / n)


# speedup vs baseline: 150.1268x; 1.6977x over previous
import functools

import jax
import jax.numpy as jnp
from jax.experimental import pallas as pl
from jax.experimental.pallas import tpu as pltpu

_L = 128
_UNROLL = 8


def _iou_vreg(a, b, mask):
    r2a = pltpu.roll(a, 6, 0)
    r2b = pltpu.roll(b, 6, 0)
    r3a = pltpu.roll(r2a, 7, 0)
    r3b = pltpu.roll(r2b, 7, 0)
    lt = jnp.maximum(a, b)
    rb = jnp.minimum(a + r2a, b + r2b)
    wh = jnp.maximum(rb - lt, 0.0)
    inter = wh * pltpu.roll(wh, 7, 0)
    denom = r2a * r3a + r2b * r3b - inter
    return jnp.where(mask, inter / denom, 0.0)


def _iou_body(a_ref, b_ref, o_ref, acc_ref, *, steps, block_groups):
    j = pl.program_id(1)
    groups_per_iter = 2 * _UNROLL
    row = jax.lax.broadcasted_iota(jnp.int32, (8, _L), 0)
    mask = (row & 3) == 0

    def body(i, acc):
        vals = []
        for k in range(_UNROLL):
            idx = i * _UNROLL + k
            a = a_ref[idx, :, :]
            b = b_ref[idx, :, :]
            vals.append(_iou_vreg(a, b, mask))
        for v in vals:
            acc = acc + v
        return acc

    acc = jax.lax.fori_loop(
        0, block_groups // groups_per_iter, body,
        jnp.zeros((8, _L), jnp.float32))

    @pl.when(j == 0)
    def _():
        acc_ref[...] = acc

    @pl.when(j > 0)
    def _():
        acc_ref[...] = acc_ref[...] + acc

    @pl.when(j == steps - 1)
    def _():
        o_ref[...] = jnp.sum(acc_ref[...], axis=0, keepdims=True).reshape(
            1, 1, _L)


@jax.jit
def kernel(boxes_pred, boxes):
    n = boxes_pred.shape[0]
    groups = n // _L
    a3 = boxes_pred.T.reshape(4, groups, _L).transpose(1, 0, 2).reshape(
        groups // 2, 8, _L)
    b3 = boxes.T.reshape(4, groups, _L).transpose(1, 0, 2).reshape(
        groups // 2, 8, _L)
    cores = 2
    block_groups = 2048
    block_rows = block_groups // 2
    steps = groups // (cores * block_groups)
    body = functools.partial(_iou_body, steps=steps,
                             block_groups=block_groups)
    partials = pl.pallas_call(
        body,
        grid=(cores, steps),
        in_specs=[
            pl.BlockSpec((block_rows, 8, _L),
                         lambda i, j, s=steps: (i * s + j, 0, 0)),
            pl.BlockSpec((block_rows, 8, _L),
                         lambda i, j, s=steps: (i * s + j, 0, 0)),
        ],
        out_specs=pl.BlockSpec((1, 1, _L), lambda i, j: (i, 0, 0)),
        out_shape=jax.ShapeDtypeStruct((cores, 1, _L), jnp.float32),
        scratch_shapes=[pltpu.VMEM((8, _L), jnp.float32)],
        compiler_params=pltpu.CompilerParams(
            dimension_semantics=("parallel", "arbitrary"),
        ),
    )(a3, b3)
    return -jnp.log(jnp.sum(partials) / n)


# speedup vs baseline: 176.8275x; 1.1779x over previous
import functools

import jax
import jax.numpy as jnp
from jax.experimental import pallas as pl
from jax.experimental.pallas import tpu as pltpu

_L = 128
_UNROLL = 16


def _iou_vreg(a, b, mask):
    r2a = pltpu.roll(a, 6, 0)
    r2b = pltpu.roll(b, 6, 0)
    r3a = pltpu.roll(r2a, 7, 0)
    r3b = pltpu.roll(r2b, 7, 0)
    lt = jnp.maximum(a, b)
    rb = jnp.minimum(a + r2a, b + r2b)
    wh = jnp.maximum(rb - lt, 0.0)
    inter = wh * pltpu.roll(wh, 7, 0)
    denom = r2a * r3a + r2b * r3b - inter
    return jnp.where(mask, inter / denom, 0.0)


def _iou_body(a_ref, b_ref, o_ref, acc_ref, *, steps, block_groups):
    j = pl.program_id(1)
    groups_per_iter = 2 * _UNROLL
    row = jax.lax.broadcasted_iota(jnp.int32, (8, _L), 0)
    mask = (row & 3) == 0

    def body(i, acc):
        vals = []
        for k in range(_UNROLL):
            idx = i * _UNROLL + k
            a = a_ref[idx, :, :]
            b = b_ref[idx, :, :]
            vals.append(_iou_vreg(a, b, mask))
        for v in vals:
            acc = acc + v
        return acc

    acc = jax.lax.fori_loop(
        0, block_groups // groups_per_iter, body,
        jnp.zeros((8, _L), jnp.float32))

    @pl.when(j == 0)
    def _():
        acc_ref[...] = acc

    @pl.when(j > 0)
    def _():
        acc_ref[...] = acc_ref[...] + acc

    @pl.when(j == steps - 1)
    def _():
        o_ref[...] = jnp.sum(acc_ref[...], axis=0, keepdims=True).reshape(
            1, 1, _L)


@jax.jit
def kernel(boxes_pred, boxes):
    n = boxes_pred.shape[0]
    groups = n // _L
    a3 = boxes_pred.T.reshape(4, groups, _L).transpose(1, 0, 2).reshape(
        groups // 2, 8, _L)
    b3 = boxes.T.reshape(4, groups, _L).transpose(1, 0, 2).reshape(
        groups // 2, 8, _L)
    cores = 2
    block_groups = 2048
    block_rows = block_groups // 2
    steps = groups // (cores * block_groups)
    body = functools.partial(_iou_body, steps=steps,
                             block_groups=block_groups)
    partials = pl.pallas_call(
        body,
        grid=(cores, steps),
        in_specs=[
            pl.BlockSpec((block_rows, 8, _L),
                         lambda i, j, s=steps: (i * s + j, 0, 0)),
            pl.BlockSpec((block_rows, 8, _L),
                         lambda i, j, s=steps: (i * s + j, 0, 0)),
        ],
        out_specs=pl.BlockSpec((1, 1, _L), lambda i, j: (i, 0, 0)),
        out_shape=jax.ShapeDtypeStruct((cores, 1, _L), jnp.float32),
        scratch_shapes=[pltpu.VMEM((8, _L), jnp.float32)],
        compiler_params=pltpu.CompilerParams(
            dimension_semantics=("parallel", "arbitrary"),
        ),
    )(a3, b3)
    return -jnp.log(jnp.sum(partials) / n)


# speedup vs baseline: 184.9321x; 1.0458x over previous
import functools

import jax
import jax.numpy as jnp
from jax.experimental import pallas as pl
from jax.experimental.pallas import tpu as pltpu

_L = 128
_UNROLL = 32


def _iou_vreg(a, b, mask):
    r2a = pltpu.roll(a, 6, 0)
    r2b = pltpu.roll(b, 6, 0)
    r3a = pltpu.roll(r2a, 7, 0)
    r3b = pltpu.roll(r2b, 7, 0)
    lt = jnp.maximum(a, b)
    rb = jnp.minimum(a + r2a, b + r2b)
    wh = jnp.maximum(rb - lt, 0.0)
    inter = wh * pltpu.roll(wh, 7, 0)
    denom = r2a * r3a + r2b * r3b - inter
    return jnp.where(mask, inter / denom, 0.0)


def _iou_body(a_ref, b_ref, o_ref, acc_ref, *, steps, block_groups):
    j = pl.program_id(1)
    groups_per_iter = 2 * _UNROLL
    row = jax.lax.broadcasted_iota(jnp.int32, (8, _L), 0)
    mask = (row & 3) == 0

    def body(i, acc):
        vals = []
        for k in range(_UNROLL):
            idx = i * _UNROLL + k
            a = a_ref[idx, :, :]
            b = b_ref[idx, :, :]
            vals.append(_iou_vreg(a, b, mask))
        for v in vals:
            acc = acc + v
        return acc

    acc = jax.lax.fori_loop(
        0, block_groups // groups_per_iter, body,
        jnp.zeros((8, _L), jnp.float32))

    @pl.when(j == 0)
    def _():
        acc_ref[...] = acc

    @pl.when(j > 0)
    def _():
        acc_ref[...] = acc_ref[...] + acc

    @pl.when(j == steps - 1)
    def _():
        o_ref[...] = jnp.sum(acc_ref[...], axis=0, keepdims=True).reshape(
            1, 1, _L)


@jax.jit
def kernel(boxes_pred, boxes):
    n = boxes_pred.shape[0]
    groups = n // _L
    a3 = boxes_pred.T.reshape(4, groups, _L).transpose(1, 0, 2).reshape(
        groups // 2, 8, _L)
    b3 = boxes.T.reshape(4, groups, _L).transpose(1, 0, 2).reshape(
        groups // 2, 8, _L)
    cores = 2
    block_groups = 2048
    block_rows = block_groups // 2
    steps = groups // (cores * block_groups)
    body = functools.partial(_iou_body, steps=steps,
                             block_groups=block_groups)
    partials = pl.pallas_call(
        body,
        grid=(cores, steps),
        in_specs=[
            pl.BlockSpec((block_rows, 8, _L),
                         lambda i, j, s=steps: (i * s + j, 0, 0)),
            pl.BlockSpec((block_rows, 8, _L),
                         lambda i, j, s=steps: (i * s + j, 0, 0)),
        ],
        out_specs=pl.BlockSpec((1, 1, _L), lambda i, j: (i, 0, 0)),
        out_shape=jax.ShapeDtypeStruct((cores, 1, _L), jnp.float32),
        scratch_shapes=[pltpu.VMEM((8, _L), jnp.float32)],
        compiler_params=pltpu.CompilerParams(
            dimension_semantics=("parallel", "arbitrary"),
        ),
    )(a3, b3)
    return -jnp.log(jnp.sum(partials) / n)


# speedup vs baseline: 193.0136x; 1.0437x over previous
import functools

import jax
import jax.numpy as jnp
from jax.experimental import pallas as pl
from jax.experimental.pallas import tpu as pltpu

_L = 128
_UNROLL = 64


def _iou_vreg(a, b):
    r2a = pltpu.roll(a, 6, 0)
    r2b = pltpu.roll(b, 6, 0)
    r3a = pltpu.roll(r2a, 7, 0)
    r3b = pltpu.roll(r2b, 7, 0)
    lt = jnp.maximum(a, b)
    rb = jnp.minimum(a + r2a, b + r2b)
    wh = jnp.maximum(rb - lt, 0.0)
    inter = wh * pltpu.roll(wh, 7, 0)
    denom = r2a * r3a + r2b * r3b - inter
    return inter / denom


def _iou_body(a_ref, b_ref, o_ref, acc_ref, *, steps, block_groups):
    j = pl.program_id(1)
    groups_per_iter = 2 * _UNROLL
    row = jax.lax.broadcasted_iota(jnp.int32, (8, _L), 0)
    mask = (row & 3) == 0

    def body(i, acc):
        vals = []
        for k in range(_UNROLL):
            idx = i * _UNROLL + k
            a = a_ref[idx, :, :]
            b = b_ref[idx, :, :]
            vals.append(_iou_vreg(a, b))
        for v in vals:
            acc = acc + v
        return acc

    acc = jax.lax.fori_loop(
        0, block_groups // groups_per_iter, body,
        jnp.zeros((8, _L), jnp.float32))

    @pl.when(j == 0)
    def _():
        acc_ref[...] = acc

    @pl.when(j > 0)
    def _():
        acc_ref[...] = acc_ref[...] + acc

    @pl.when(j == steps - 1)
    def _():
        valid = jnp.where(mask, acc_ref[...], 0.0)
        o_ref[...] = jnp.sum(valid, axis=0, keepdims=True).reshape(
            1, 1, _L)


@jax.jit
def kernel(boxes_pred, boxes):
    n = boxes_pred.shape[0]
    groups = n // _L
    a3 = boxes_pred.T.reshape(4, groups, _L).transpose(1, 0, 2).reshape(
        groups // 2, 8, _L)
    b3 = boxes.T.reshape(4, groups, _L).transpose(1, 0, 2).reshape(
        groups // 2, 8, _L)
    cores = 2
    block_groups = 2048
    block_rows = block_groups // 2
    steps = groups // (cores * block_groups)
    body = functools.partial(_iou_body, steps=steps,
                             block_groups=block_groups)
    partials = pl.pallas_call(
        body,
        grid=(cores, steps),
        in_specs=[
            pl.BlockSpec((block_rows, 8, _L),
                         lambda i, j, s=steps: (i * s + j, 0, 0)),
            pl.BlockSpec((block_rows, 8, _L),
                         lambda i, j, s=steps: (i * s + j, 0, 0)),
        ],
        out_specs=pl.BlockSpec((1, 1, _L), lambda i, j: (i, 0, 0)),
        out_shape=jax.ShapeDtypeStruct((cores, 1, _L), jnp.float32),
        scratch_shapes=[pltpu.VMEM((8, _L), jnp.float32)],
        compiler_params=pltpu.CompilerParams(
            dimension_semantics=("parallel", "arbitrary"),
        ),
    )(a3, b3)
    return -jnp.log(jnp.sum(partials) / n)


# speedup vs baseline: 205.9452x; 1.0670x over previous
import functools

import jax
import jax.numpy as jnp
from jax.experimental import pallas as pl
from jax.experimental.pallas import tpu as pltpu

_L = 128
_UNROLL = 64


def _iou_vreg(a, b):
    r2a = pltpu.roll(a, 6, 0)
    r2b = pltpu.roll(b, 6, 0)
    r3a = pltpu.roll(r2a, 7, 0)
    r3b = pltpu.roll(r2b, 7, 0)
    lt = jnp.maximum(a, b)
    rb = jnp.minimum(a + r2a, b + r2b)
    wh = jnp.maximum(rb - lt, 0.0)
    inter = wh * pltpu.roll(wh, 7, 0)
    denom = r2a * r3a + r2b * r3b - inter
    return inter / denom


def _iou_body(a_ref, b_ref, o_ref, acc_ref, *, steps, block_groups):
    j = pl.program_id(1)
    groups_per_iter = 2 * _UNROLL
    row = jax.lax.broadcasted_iota(jnp.int32, (8, _L), 0)
    mask = (row & 3) == 0

    def body(i, acc):
        vals = []
        for k in range(_UNROLL):
            idx = i * _UNROLL + k
            a = a_ref[idx, :, :]
            b = b_ref[idx, :, :]
            vals.append(_iou_vreg(a, b))
        for v in vals:
            acc = acc + v
        return acc

    acc = jax.lax.fori_loop(
        0, block_groups // groups_per_iter, body,
        jnp.zeros((8, _L), jnp.float32))

    @pl.when(j == 0)
    def _():
        acc_ref[...] = acc

    @pl.when(j > 0)
    def _():
        acc_ref[...] = acc_ref[...] + acc

    @pl.when(j == steps - 1)
    def _():
        valid = jnp.where(mask, acc_ref[...], 0.0)
        o_ref[...] = jnp.sum(valid, axis=0, keepdims=True).reshape(
            1, 1, _L)


@jax.jit
def kernel(boxes_pred, boxes):
    n = boxes_pred.shape[0]
    groups = n // _L
    a3 = boxes_pred.T.reshape(4, groups, _L).transpose(1, 0, 2).reshape(
        groups // 2, 8, _L)
    b3 = boxes.T.reshape(4, groups, _L).transpose(1, 0, 2).reshape(
        groups // 2, 8, _L)
    cores = 2
    block_groups = 4096
    block_rows = block_groups // 2
    steps = groups // (cores * block_groups)
    body = functools.partial(_iou_body, steps=steps,
                             block_groups=block_groups)
    partials = pl.pallas_call(
        body,
        grid=(cores, steps),
        in_specs=[
            pl.BlockSpec((block_rows, 8, _L),
                         lambda i, j, s=steps: (i * s + j, 0, 0)),
            pl.BlockSpec((block_rows, 8, _L),
                         lambda i, j, s=steps: (i * s + j, 0, 0)),
        ],
        out_specs=pl.BlockSpec((1, 1, _L), lambda i, j: (i, 0, 0)),
        out_shape=jax.ShapeDtypeStruct((cores, 1, _L), jnp.float32),
        scratch_shapes=[pltpu.VMEM((8, _L), jnp.float32)],
        compiler_params=pltpu.CompilerParams(
            dimension_semantics=("parallel", "arbitrary"),
        ),
    )(a3, b3)
    return -jnp.log(jnp.sum(partials) / n)
